# baseline (speedup 1.0000x reference)
import functools

import jax
import jax.numpy as jnp
from jax import lax
from jax.experimental import pallas as pl
from jax.experimental.pallas import tpu as pltpu
from jax.experimental.pallas import tpu_sc as plsc

N = 10000
D_IN = 128
HIDDEN = 128
D = 64
E = 320000

NC = 2
NS = 16
N_PAD = 10240
CHUNK = 128
E_PAD = 327680
R = E_PAD // CHUNK
ROWS_SUB = N_PAD // NS
KD = 6
KG = 3
RT0 = 28
RT1 = 132
RT_MAX = max(RT0, RT1)
RT_EQ = R // (NC * NS)

_mesh = plsc.VectorSubcoreMesh(core_axis_name="c", subcore_axis_name="s")


@functools.partial(
    pl.kernel,
    mesh=_mesh,
    out_type=jax.ShapeDtypeStruct((NC * N_PAD, D), jnp.float32),
    compiler_params=pltpu.CompilerParams(use_tc_tiling_on_sc=False),
    scratch_types=[
        pltpu.VMEM((RT_MAX, CHUNK), jnp.int32),
        pltpu.VMEM((RT_MAX, CHUNK), jnp.int32),
        pltpu.VMEM((KD, CHUNK, D), jnp.float32),
        pltpu.SemaphoreType.DMA,
        pltpu.SemaphoreType.DMA,
        pltpu.VMEM_SHARED((N_PAD, D), jnp.float32),
    ],
)
def _sc_spmm(u_hbm, src_hbm, dst_hbm, out_hbm, src_v, dst_v, ring_v, gsem, ssem, acc_sh):
    c = lax.axis_index("c")
    s = lax.axis_index("s")
    base = s * ROWS_SUB
    n_rows = jnp.where(c == 0, RT0, RT1)
    tile_base = jnp.where(c == 0, s * RT0, NS * RT0 + s * RT1)

    pltpu.sync_copy(src_hbm.at[pl.ds(tile_base, RT_MAX)], src_v)
    pltpu.sync_copy(dst_hbm.at[pl.ds(tile_base, RT_MAX)], dst_v)

    zero16 = jnp.zeros((16,), jnp.float32)
    for i in range(CHUNK):
        for j in range(D // 16):
            ring_v[0, i, pl.ds(j * 16, 16)] = zero16
    for k in range(ROWS_SUB // CHUNK):
        pltpu.sync_copy(ring_v.at[0], acc_sh.at[pl.ds(base + k * CHUNK, CHUNK)])
    plsc.subcore_barrier()

    def pstep(g, _):
        for b in range(KD):
            r = g * KD + b

            @pl.when(jnp.logical_and(r >= KD, r - KD < n_rows))
            def _():
                pltpu.make_async_copy(
                    ring_v.at[b], acc_sh.at[dst_v.at[r - KD]], ssem).wait()

            @pl.when(r < n_rows)
            def _():
                pltpu.async_copy(u_hbm.at[src_v.at[r]], ring_v.at[b], gsem)

            bg = (b - KG) % KD

            @pl.when(jnp.logical_and(r >= KG, r - KG < n_rows))
            def _():
                pltpu.make_async_copy(
                    u_hbm.at[src_v.at[r - KG]], ring_v.at[bg], gsem).wait()
                pltpu.async_copy(
                    ring_v.at[bg], acc_sh.at[dst_v.at[r - KG]], ssem, add=True)
        return 0

    lax.fori_loop(0, RT_MAX // KD + 2, pstep, 0)
    plsc.subcore_barrier()

    def obody(k, _):
        off = base + k * CHUNK
        pltpu.sync_copy(acc_sh.at[pl.ds(off, CHUNK)], ring_v.at[0])
        pltpu.sync_copy(ring_v.at[0], out_hbm.at[pl.ds(c * N_PAD + off, CHUNK)])
        return 0

    lax.fori_loop(0, ROWS_SUB // CHUNK, obody, 0)


@functools.partial(
    pl.kernel,
    mesh=_mesh,
    out_type=jax.ShapeDtypeStruct((NC * N_PAD,), jnp.float32),
    scratch_types=[
        pltpu.VMEM((RT_EQ, CHUNK), jnp.int32),
        pltpu.VMEM((1, CHUNK), jnp.float32),
        pltpu.VMEM((ROWS_SUB,), jnp.float32),
        pltpu.SemaphoreType.DMA,
        pltpu.VMEM_SHARED((N_PAD,), jnp.float32),
    ],
)
def _sc_degree(src_hbm, out_hbm, src_v, ones_v, buf_v, ssem, acc_sh):
    c = lax.axis_index("c")
    s = lax.axis_index("s")

    one16 = jnp.ones((16,), jnp.float32)
    zero16 = jnp.zeros((16,), jnp.float32)
    for j in range(CHUNK // 16):
        ones_v[0, pl.ds(j * 16, 16)] = one16

    def zb(k, _):
        buf_v[pl.ds(k * 16, 16)] = zero16
        return 0

    lax.fori_loop(0, ROWS_SUB // 16, zb, 0)
    base = s * ROWS_SUB
    tile_base = (c * NS + s) * RT_EQ
    pltpu.sync_copy(src_hbm.at[pl.ds(tile_base, RT_EQ)], src_v)
    pltpu.sync_copy(buf_v, acc_sh.at[pl.ds(base, ROWS_SUB)])
    plsc.subcore_barrier()

    GRP = 16

    def ebody(g, _):
        for b in range(GRP):
            pltpu.async_copy(
                ones_v.at[0], acc_sh.at[src_v.at[g * GRP + b]], ssem, add=True)
        for b in range(GRP):
            pltpu.make_async_copy(
                ones_v.at[0], acc_sh.at[src_v.at[g * GRP + b]], ssem).wait()
        return 0

    lax.fori_loop(0, RT_EQ // GRP, ebody, 0)
    plsc.subcore_barrier()

    pltpu.sync_copy(acc_sh.at[pl.ds(base, ROWS_SUB)], buf_v)
    pltpu.sync_copy(buf_v, out_hbm.at[pl.ds(c * N_PAD + base, ROWS_SUB)])


def _t1_body(x_ref, w1_ref, b1_ref, cw_ref, degp_ref, p0_ref, p1_ref, p2_ref, dis_ref):
    degp = degp_ref[...]
    deg = jnp.reshape(degp[0] + degp[1], (N_PAD, 1))
    rid = lax.broadcasted_iota(jnp.int32, (N_PAD, 1), 0)
    dis = jnp.where((rid < N) & (deg > 0.0),
                    lax.rsqrt(jnp.maximum(deg, 1e-12)), 0.0)
    dis_ref[...] = dis
    h = jnp.maximum(
        jnp.dot(x_ref[...], w1_ref[...], preferred_element_type=jnp.float32)
        + b1_ref[...], 0.0)
    cw = cw_ref[...]
    p0_ref[...] = jnp.dot(h, cw[0] - cw[2], preferred_element_type=jnp.float32)
    p1_ref[...] = jnp.dot(h, cw[1], preferred_element_type=jnp.float32)
    p2_ref[...] = dis * jnp.dot(h, cw[2], preferred_element_type=jnp.float32)


_t1 = pl.pallas_call(
    _t1_body,
    out_shape=[
        jax.ShapeDtypeStruct((N_PAD, D), jnp.float32),
        jax.ShapeDtypeStruct((N_PAD, D), jnp.float32),
        jax.ShapeDtypeStruct((N_PAD, D), jnp.float32),
        jax.ShapeDtypeStruct((N_PAD, 1), jnp.float32),
    ],
)


def _tmid_body(p1_ref, q_ref, dis_ref, z_ref):
    q = q_ref[...]
    dis = dis_ref[...]
    z_ref[...] = dis * p1_ref[...] - 2.0 * dis * dis * (q[0] + q[1])


_tmid = pl.pallas_call(
    _tmid_body,
    out_shape=jax.ShapeDtypeStruct((N_PAD, D), jnp.float32),
)


def _t3_body(p0_ref, q_ref, dis_ref, bc_ref, cw_ref, r0_ref, r1_ref, r2_ref):
    q = q_ref[...]
    dis = dis_ref[...]
    g = jnp.maximum(p0_ref[...] - dis * (q[0] + q[1]) + bc_ref[...], 0.0)
    cw = cw_ref[...]
    r0_ref[...] = jnp.dot(g, cw[0] - cw[2], preferred_element_type=jnp.float32)
    r1_ref[...] = jnp.dot(g, cw[1], preferred_element_type=jnp.float32)
    r2_ref[...] = dis * jnp.dot(g, cw[2], preferred_element_type=jnp.float32)


_t3 = pl.pallas_call(
    _t3_body,
    out_shape=[
        jax.ShapeDtypeStruct((N_PAD, D), jnp.float32),
        jax.ShapeDtypeStruct((N_PAD, D), jnp.float32),
        jax.ShapeDtypeStruct((N_PAD, D), jnp.float32),
    ],
)


def _t5_body(r0_ref, q_ref, dis_ref, bc_ref, w2_ref, b2_ref, out_ref):
    q = q_ref[...]
    dis = dis_ref[...]
    f = jnp.maximum(r0_ref[...] - dis * (q[0] + q[1]) + bc_ref[...], 0.0)
    logits = jnp.dot(f, w2_ref[...], preferred_element_type=jnp.float32) + b2_ref[...]
    col = lax.broadcasted_iota(jnp.int32, (N_PAD, HIDDEN), 1)
    mask = col < 2
    ml = jnp.where(mask, logits, -jnp.inf)
    m = jnp.max(ml, axis=1, keepdims=True)
    e = jnp.where(mask, jnp.exp(logits - m), 0.0)
    out_ref[...] = e / jnp.sum(e, axis=1, keepdims=True)


_t5 = pl.pallas_call(
    _t5_body,
    out_shape=jax.ShapeDtypeStruct((N_PAD, HIDDEN), jnp.float32),
)


@jax.jit
def kernel(x, edge_index, lin1_W, lin1_b, conv1_W, conv1_b, conv2_W, conv2_b,
           lin2_W, lin2_b):
    x_pad = jnp.zeros((N_PAD, D_IN), jnp.float32).at[:N].set(x)
    src = jnp.full((E_PAD,), N, jnp.int32).at[:E].set(
        edge_index[0].astype(jnp.int32)).reshape(R, CHUNK)
    dst = jnp.full((E_PAD,), N, jnp.int32).at[:E].set(
        edge_index[1].astype(jnp.int32)).reshape(R, CHUNK)
    w2_pad = jnp.zeros((D, HIDDEN), jnp.float32).at[:, :2].set(lin2_W)
    b2_pad = jnp.zeros((1, HIDDEN), jnp.float32).at[:, :2].set(lin2_b)

    degp = _sc_degree(src).reshape(NC, N_PAD)
    p0, p1, p2, dis = _t1(x_pad, lin1_W, lin1_b.reshape(1, HIDDEN), conv1_W, degp)

    q1 = _sc_spmm(p2, src, dst).reshape(NC, N_PAD, D)
    z1 = _tmid(p1, q1, dis)
    q2 = _sc_spmm(z1, src, dst).reshape(NC, N_PAD, D)
    r0, r1, r2 = _t3(p0, q2, dis, conv1_b.reshape(1, D), conv2_W)

    q3 = _sc_spmm(r2, src, dst).reshape(NC, N_PAD, D)
    z2 = _tmid(r1, q3, dis)
    q4 = _sc_spmm(z2, src, dst).reshape(NC, N_PAD, D)
    out = _t5(r0, q4, dis, conv2_b.reshape(1, D), w2_pad, b2_pad)

    return out[:N, :2]


# speedup vs baseline: 1.0844x; 1.0844x over previous
import functools

import jax
import jax.numpy as jnp
from jax import lax
from jax.experimental import pallas as pl
from jax.experimental.pallas import tpu as pltpu
from jax.experimental.pallas import tpu_sc as plsc

N = 10000
D_IN = 128
HIDDEN = 128
D = 64
E = 320000

NC = 2
NS = 16
N_PAD = 10240
CHUNK = 128
E_PAD = 327680
R = E_PAD // CHUNK
ROWS_SUB = N_PAD // NS
KD = 6
KG = 3
RT0 = 132
RT1 = 28
RT_MAX = max(RT0, RT1)
RT_EQ = R // (NC * NS)

_mesh = plsc.VectorSubcoreMesh(core_axis_name="c", subcore_axis_name="s")


@functools.partial(
    pl.kernel,
    mesh=_mesh,
    out_type=jax.ShapeDtypeStruct((NC * N_PAD, D), jnp.float32),
    compiler_params=pltpu.CompilerParams(use_tc_tiling_on_sc=False),
    scratch_types=[
        pltpu.VMEM((RT_MAX, CHUNK), jnp.int32),
        pltpu.VMEM((RT_MAX, CHUNK), jnp.int32),
        pltpu.VMEM((KD, CHUNK, D), jnp.float32),
        pltpu.SemaphoreType.DMA,
        pltpu.SemaphoreType.DMA,
        pltpu.VMEM_SHARED((N_PAD, D), jnp.float32),
    ],
)
def _sc_spmm(u_hbm, src_hbm, dst_hbm, out_hbm, src_v, dst_v, ring_v, gsem, ssem, acc_sh):
    c = lax.axis_index("c")
    s = lax.axis_index("s")
    base = s * ROWS_SUB
    n_rows = jnp.where(c == 0, RT0, RT1)
    tile_base = jnp.where(c == 0, s * RT0, NS * RT0 + s * RT1)

    pltpu.sync_copy(src_hbm.at[pl.ds(tile_base, RT_MAX)], src_v)
    pltpu.sync_copy(dst_hbm.at[pl.ds(tile_base, RT_MAX)], dst_v)

    zero16 = jnp.zeros((16,), jnp.float32)
    for i in range(CHUNK):
        for j in range(D // 16):
            ring_v[0, i, pl.ds(j * 16, 16)] = zero16
    for k in range(ROWS_SUB // CHUNK):
        pltpu.sync_copy(ring_v.at[0], acc_sh.at[pl.ds(base + k * CHUNK, CHUNK)])
    plsc.subcore_barrier()

    def pstep(g, _):
        for b in range(KD):
            r = g * KD + b

            @pl.when(jnp.logical_and(r >= KD, r - KD < n_rows))
            def _():
                pltpu.make_async_copy(
                    ring_v.at[b], acc_sh.at[dst_v.at[r - KD]], ssem).wait()

            @pl.when(r < n_rows)
            def _():
                pltpu.async_copy(u_hbm.at[src_v.at[r]], ring_v.at[b], gsem)

            bg = (b - KG) % KD

            @pl.when(jnp.logical_and(r >= KG, r - KG < n_rows))
            def _():
                pltpu.make_async_copy(
                    u_hbm.at[src_v.at[r - KG]], ring_v.at[bg], gsem).wait()
                pltpu.async_copy(
                    ring_v.at[bg], acc_sh.at[dst_v.at[r - KG]], ssem, add=True)
        return 0

    lax.fori_loop(0, RT_MAX // KD + 2, pstep, 0)
    plsc.subcore_barrier()

    def obody(k, _):
        off = base + k * CHUNK
        pltpu.sync_copy(acc_sh.at[pl.ds(off, CHUNK)], ring_v.at[0])
        pltpu.sync_copy(ring_v.at[0], out_hbm.at[pl.ds(c * N_PAD + off, CHUNK)])
        return 0

    lax.fori_loop(0, ROWS_SUB // CHUNK, obody, 0)


@functools.partial(
    pl.kernel,
    mesh=_mesh,
    out_type=jax.ShapeDtypeStruct((NC * N_PAD,), jnp.float32),
    scratch_types=[
        pltpu.VMEM((RT_EQ, CHUNK), jnp.int32),
        pltpu.VMEM((1, CHUNK), jnp.float32),
        pltpu.VMEM((ROWS_SUB,), jnp.float32),
        pltpu.SemaphoreType.DMA,
        pltpu.VMEM_SHARED((N_PAD,), jnp.float32),
    ],
)
def _sc_degree(src_hbm, out_hbm, src_v, ones_v, buf_v, ssem, acc_sh):
    c = lax.axis_index("c")
    s = lax.axis_index("s")

    one16 = jnp.ones((16,), jnp.float32)
    zero16 = jnp.zeros((16,), jnp.float32)
    for j in range(CHUNK // 16):
        ones_v[0, pl.ds(j * 16, 16)] = one16

    def zb(k, _):
        buf_v[pl.ds(k * 16, 16)] = zero16
        return 0

    lax.fori_loop(0, ROWS_SUB // 16, zb, 0)
    base = s * ROWS_SUB
    tile_base = (c * NS + s) * RT_EQ
    pltpu.sync_copy(src_hbm.at[pl.ds(tile_base, RT_EQ)], src_v)
    pltpu.sync_copy(buf_v, acc_sh.at[pl.ds(base, ROWS_SUB)])
    plsc.subcore_barrier()

    GRP = 16

    def ebody(g, _):
        for b in range(GRP):
            pltpu.async_copy(
                ones_v.at[0], acc_sh.at[src_v.at[g * GRP + b]], ssem, add=True)
        for b in range(GRP):
            pltpu.make_async_copy(
                ones_v.at[0], acc_sh.at[src_v.at[g * GRP + b]], ssem).wait()
        return 0

    lax.fori_loop(0, RT_EQ // GRP, ebody, 0)
    plsc.subcore_barrier()

    pltpu.sync_copy(acc_sh.at[pl.ds(base, ROWS_SUB)], buf_v)
    pltpu.sync_copy(buf_v, out_hbm.at[pl.ds(c * N_PAD + base, ROWS_SUB)])


def _t1_body(x_ref, w1_ref, b1_ref, cw_ref, degp_ref, p0_ref, p1_ref, p2_ref, dis_ref):
    degp = degp_ref[...]
    deg = jnp.reshape(degp[0] + degp[1], (N_PAD, 1))
    rid = lax.broadcasted_iota(jnp.int32, (N_PAD, 1), 0)
    dis = jnp.where((rid < N) & (deg > 0.0),
                    lax.rsqrt(jnp.maximum(deg, 1e-12)), 0.0)
    dis_ref[...] = dis
    h = jnp.maximum(
        jnp.dot(x_ref[...], w1_ref[...], preferred_element_type=jnp.float32)
        + b1_ref[...], 0.0)
    cw = cw_ref[...]
    p0_ref[...] = jnp.dot(h, cw[0] - cw[2], preferred_element_type=jnp.float32)
    p1_ref[...] = jnp.dot(h, cw[1], preferred_element_type=jnp.float32)
    p2_ref[...] = dis * jnp.dot(h, cw[2], preferred_element_type=jnp.float32)


_t1 = pl.pallas_call(
    _t1_body,
    out_shape=[
        jax.ShapeDtypeStruct((N_PAD, D), jnp.float32),
        jax.ShapeDtypeStruct((N_PAD, D), jnp.float32),
        jax.ShapeDtypeStruct((N_PAD, D), jnp.float32),
        jax.ShapeDtypeStruct((N_PAD, 1), jnp.float32),
    ],
)


def _tmid_body(p1_ref, q_ref, dis_ref, z_ref):
    q = q_ref[...]
    dis = dis_ref[...]
    z_ref[...] = dis * p1_ref[...] - 2.0 * dis * dis * (q[0] + q[1])


_tmid = pl.pallas_call(
    _tmid_body,
    out_shape=jax.ShapeDtypeStruct((N_PAD, D), jnp.float32),
)


def _t3_body(p0_ref, q_ref, dis_ref, bc_ref, cw_ref, r0_ref, r1_ref, r2_ref):
    q = q_ref[...]
    dis = dis_ref[...]
    g = jnp.maximum(p0_ref[...] - dis * (q[0] + q[1]) + bc_ref[...], 0.0)
    cw = cw_ref[...]
    r0_ref[...] = jnp.dot(g, cw[0] - cw[2], preferred_element_type=jnp.float32)
    r1_ref[...] = jnp.dot(g, cw[1], preferred_element_type=jnp.float32)
    r2_ref[...] = dis * jnp.dot(g, cw[2], preferred_element_type=jnp.float32)


_t3 = pl.pallas_call(
    _t3_body,
    out_shape=[
        jax.ShapeDtypeStruct((N_PAD, D), jnp.float32),
        jax.ShapeDtypeStruct((N_PAD, D), jnp.float32),
        jax.ShapeDtypeStruct((N_PAD, D), jnp.float32),
    ],
)


def _t5_body(r0_ref, q_ref, dis_ref, bc_ref, w2_ref, b2_ref, out_ref):
    q = q_ref[...]
    dis = dis_ref[...]
    f = jnp.maximum(r0_ref[...] - dis * (q[0] + q[1]) + bc_ref[...], 0.0)
    logits = jnp.dot(f, w2_ref[...], preferred_element_type=jnp.float32) + b2_ref[...]
    col = lax.broadcasted_iota(jnp.int32, (N_PAD, HIDDEN), 1)
    mask = col < 2
    ml = jnp.where(mask, logits, -jnp.inf)
    m = jnp.max(ml, axis=1, keepdims=True)
    e = jnp.where(mask, jnp.exp(logits - m), 0.0)
    out_ref[...] = e / jnp.sum(e, axis=1, keepdims=True)


_t5 = pl.pallas_call(
    _t5_body,
    out_shape=jax.ShapeDtypeStruct((N_PAD, HIDDEN), jnp.float32),
)


@jax.jit
def kernel(x, edge_index, lin1_W, lin1_b, conv1_W, conv1_b, conv2_W, conv2_b,
           lin2_W, lin2_b):
    x_pad = jnp.zeros((N_PAD, D_IN), jnp.float32).at[:N].set(x)
    src = jnp.full((E_PAD,), N, jnp.int32).at[:E].set(
        edge_index[0].astype(jnp.int32)).reshape(R, CHUNK)
    dst = jnp.full((E_PAD,), N, jnp.int32).at[:E].set(
        edge_index[1].astype(jnp.int32)).reshape(R, CHUNK)
    w2_pad = jnp.zeros((D, HIDDEN), jnp.float32).at[:, :2].set(lin2_W)
    b2_pad = jnp.zeros((1, HIDDEN), jnp.float32).at[:, :2].set(lin2_b)

    degp = _sc_degree(src).reshape(NC, N_PAD)
    p0, p1, p2, dis = _t1(x_pad, lin1_W, lin1_b.reshape(1, HIDDEN), conv1_W, degp)

    q1 = _sc_spmm(p2, src, dst).reshape(NC, N_PAD, D)
    z1 = _tmid(p1, q1, dis)
    q2 = _sc_spmm(z1, src, dst).reshape(NC, N_PAD, D)
    r0, r1, r2 = _t3(p0, q2, dis, conv1_b.reshape(1, D), conv2_W)

    q3 = _sc_spmm(r2, src, dst).reshape(NC, N_PAD, D)
    z2 = _tmid(r1, q3, dis)
    q4 = _sc_spmm(z2, src, dst).reshape(NC, N_PAD, D)
    out = _t5(r0, q4, dis, conv2_b.reshape(1, D), w2_pad, b2_pad)

    return out[:N, :2]


# speedup vs baseline: 2.4977x; 2.3034x over previous
import functools

import jax
import jax.numpy as jnp
from jax import lax
from jax.experimental import pallas as pl
from jax.experimental.pallas import tpu as pltpu
from jax.experimental.pallas import tpu_sc as plsc

N = 10000
D_IN = 128
HIDDEN = 128
D = 64
E = 320000

NC = 2
NS = 16
N_PAD = 10240
CHUNK = 64
E_PAD = 327680
R = E_PAD // CHUNK
ROWS_SUB = N_PAD // NS
KD = 6
KG = 3
RT0 = 160
RT1 = 160
RT_MAX = max(RT0, RT1)
RT_EQ = R // (NC * NS)

_mesh = plsc.VectorSubcoreMesh(core_axis_name="c", subcore_axis_name="s")


@functools.partial(
    pl.kernel,
    mesh=_mesh,
    out_type=jax.ShapeDtypeStruct((NC * N_PAD, D), jnp.float32),
    compiler_params=pltpu.CompilerParams(use_tc_tiling_on_sc=False),
    scratch_types=[
        pltpu.VMEM((RT_MAX, CHUNK), jnp.int32),
        pltpu.VMEM((RT_MAX, CHUNK), jnp.int32),
        pltpu.VMEM((KD, CHUNK, D), jnp.float32),
        pltpu.SemaphoreType.DMA,
        pltpu.SemaphoreType.DMA,
        pltpu.VMEM_SHARED((N_PAD, D), jnp.float32),
        pltpu.VMEM_SHARED((N_PAD, D), jnp.float32),
    ],
)
def _sc_spmm(u_hbm, src_hbm, dst_hbm, out_hbm, src_v, dst_v, ring_v, gsem, ssem, acc_sh, u_sh):
    c = lax.axis_index("c")
    s = lax.axis_index("s")
    base = s * ROWS_SUB
    n_rows = jnp.where(c == 0, RT0, RT1)
    tile_base = jnp.where(c == 0, s * RT0, NS * RT0 + s * RT1)

    pltpu.sync_copy(src_hbm.at[pl.ds(tile_base, RT_MAX)], src_v)
    pltpu.sync_copy(dst_hbm.at[pl.ds(tile_base, RT_MAX)], dst_v)
    pltpu.sync_copy(u_hbm.at[pl.ds(base, ROWS_SUB)], u_sh.at[pl.ds(base, ROWS_SUB)])

    zero16 = jnp.zeros((16,), jnp.float32)
    for i in range(CHUNK):
        for j in range(D // 16):
            ring_v[0, i, pl.ds(j * 16, 16)] = zero16
    for k in range(ROWS_SUB // CHUNK):
        pltpu.sync_copy(ring_v.at[0], acc_sh.at[pl.ds(base + k * CHUNK, CHUNK)])
    plsc.subcore_barrier()

    def pstep(g, _):
        for b in range(KD):
            r = g * KD + b

            @pl.when(jnp.logical_and(r >= KD, r - KD < n_rows))
            def _():
                pltpu.make_async_copy(
                    ring_v.at[b], acc_sh.at[dst_v.at[r - KD]], ssem).wait()

            @pl.when(r < n_rows)
            def _():
                pltpu.async_copy(u_sh.at[src_v.at[r]], ring_v.at[b], gsem)

            bg = (b - KG) % KD

            @pl.when(jnp.logical_and(r >= KG, r - KG < n_rows))
            def _():
                pltpu.make_async_copy(
                    u_sh.at[src_v.at[r - KG]], ring_v.at[bg], gsem).wait()
                pltpu.async_copy(
                    ring_v.at[bg], acc_sh.at[dst_v.at[r - KG]], ssem, add=True)
        return 0

    lax.fori_loop(0, RT_MAX // KD + 2, pstep, 0)
    plsc.subcore_barrier()

    def obody(k, _):
        off = base + k * CHUNK
        pltpu.sync_copy(acc_sh.at[pl.ds(off, CHUNK)], ring_v.at[0])
        pltpu.sync_copy(ring_v.at[0], out_hbm.at[pl.ds(c * N_PAD + off, CHUNK)])
        return 0

    lax.fori_loop(0, ROWS_SUB // CHUNK, obody, 0)


@functools.partial(
    pl.kernel,
    mesh=_mesh,
    out_type=jax.ShapeDtypeStruct((NC * N_PAD,), jnp.float32),
    scratch_types=[
        pltpu.VMEM((RT_EQ, CHUNK), jnp.int32),
        pltpu.VMEM((1, CHUNK), jnp.float32),
        pltpu.VMEM((ROWS_SUB,), jnp.float32),
        pltpu.SemaphoreType.DMA,
        pltpu.VMEM_SHARED((N_PAD,), jnp.float32),
    ],
)
def _sc_degree(src_hbm, out_hbm, src_v, ones_v, buf_v, ssem, acc_sh):
    c = lax.axis_index("c")
    s = lax.axis_index("s")

    one16 = jnp.ones((16,), jnp.float32)
    zero16 = jnp.zeros((16,), jnp.float32)
    for j in range(CHUNK // 16):
        ones_v[0, pl.ds(j * 16, 16)] = one16

    def zb(k, _):
        buf_v[pl.ds(k * 16, 16)] = zero16
        return 0

    lax.fori_loop(0, ROWS_SUB // 16, zb, 0)
    base = s * ROWS_SUB
    tile_base = (c * NS + s) * RT_EQ
    pltpu.sync_copy(src_hbm.at[pl.ds(tile_base, RT_EQ)], src_v)
    pltpu.sync_copy(buf_v, acc_sh.at[pl.ds(base, ROWS_SUB)])
    plsc.subcore_barrier()

    GRP = 16

    def ebody(g, _):
        for b in range(GRP):
            pltpu.async_copy(
                ones_v.at[0], acc_sh.at[src_v.at[g * GRP + b]], ssem, add=True)
        for b in range(GRP):
            pltpu.make_async_copy(
                ones_v.at[0], acc_sh.at[src_v.at[g * GRP + b]], ssem).wait()
        return 0

    lax.fori_loop(0, RT_EQ // GRP, ebody, 0)
    plsc.subcore_barrier()

    pltpu.sync_copy(acc_sh.at[pl.ds(base, ROWS_SUB)], buf_v)
    pltpu.sync_copy(buf_v, out_hbm.at[pl.ds(c * N_PAD + base, ROWS_SUB)])


def _t1_body(x_ref, w1_ref, b1_ref, cw_ref, degp_ref, p0_ref, p1_ref, p2_ref, dis_ref):
    degp = degp_ref[...]
    deg = jnp.reshape(degp[0] + degp[1], (N_PAD, 1))
    rid = lax.broadcasted_iota(jnp.int32, (N_PAD, 1), 0)
    dis = jnp.where((rid < N) & (deg > 0.0),
                    lax.rsqrt(jnp.maximum(deg, 1e-12)), 0.0)
    dis_ref[...] = dis
    h = jnp.maximum(
        jnp.dot(x_ref[...], w1_ref[...], preferred_element_type=jnp.float32)
        + b1_ref[...], 0.0)
    cw = cw_ref[...]
    p0_ref[...] = jnp.dot(h, cw[0] - cw[2], preferred_element_type=jnp.float32)
    p1_ref[...] = jnp.dot(h, cw[1], preferred_element_type=jnp.float32)
    p2_ref[...] = dis * jnp.dot(h, cw[2], preferred_element_type=jnp.float32)


_t1 = pl.pallas_call(
    _t1_body,
    out_shape=[
        jax.ShapeDtypeStruct((N_PAD, D), jnp.float32),
        jax.ShapeDtypeStruct((N_PAD, D), jnp.float32),
        jax.ShapeDtypeStruct((N_PAD, D), jnp.float32),
        jax.ShapeDtypeStruct((N_PAD, 1), jnp.float32),
    ],
)


def _tmid_body(p1_ref, q_ref, dis_ref, z_ref):
    q = q_ref[...]
    dis = dis_ref[...]
    z_ref[...] = dis * p1_ref[...] - 2.0 * dis * dis * (q[0] + q[1])


_tmid = pl.pallas_call(
    _tmid_body,
    out_shape=jax.ShapeDtypeStruct((N_PAD, D), jnp.float32),
)


def _t3_body(p0_ref, q_ref, dis_ref, bc_ref, cw_ref, r0_ref, r1_ref, r2_ref):
    q = q_ref[...]
    dis = dis_ref[...]
    g = jnp.maximum(p0_ref[...] - dis * (q[0] + q[1]) + bc_ref[...], 0.0)
    cw = cw_ref[...]
    r0_ref[...] = jnp.dot(g, cw[0] - cw[2], preferred_element_type=jnp.float32)
    r1_ref[...] = jnp.dot(g, cw[1], preferred_element_type=jnp.float32)
    r2_ref[...] = dis * jnp.dot(g, cw[2], preferred_element_type=jnp.float32)


_t3 = pl.pallas_call(
    _t3_body,
    out_shape=[
        jax.ShapeDtypeStruct((N_PAD, D), jnp.float32),
        jax.ShapeDtypeStruct((N_PAD, D), jnp.float32),
        jax.ShapeDtypeStruct((N_PAD, D), jnp.float32),
    ],
)


def _t5_body(r0_ref, q_ref, dis_ref, bc_ref, w2_ref, b2_ref, out_ref):
    q = q_ref[...]
    dis = dis_ref[...]
    f = jnp.maximum(r0_ref[...] - dis * (q[0] + q[1]) + bc_ref[...], 0.0)
    logits = jnp.dot(f, w2_ref[...], preferred_element_type=jnp.float32) + b2_ref[...]
    col = lax.broadcasted_iota(jnp.int32, (N_PAD, HIDDEN), 1)
    mask = col < 2
    ml = jnp.where(mask, logits, -jnp.inf)
    m = jnp.max(ml, axis=1, keepdims=True)
    e = jnp.where(mask, jnp.exp(logits - m), 0.0)
    out_ref[...] = e / jnp.sum(e, axis=1, keepdims=True)


_t5 = pl.pallas_call(
    _t5_body,
    out_shape=jax.ShapeDtypeStruct((N_PAD, HIDDEN), jnp.float32),
)


@jax.jit
def kernel(x, edge_index, lin1_W, lin1_b, conv1_W, conv1_b, conv2_W, conv2_b,
           lin2_W, lin2_b):
    x_pad = jnp.zeros((N_PAD, D_IN), jnp.float32).at[:N].set(x)
    src = jnp.full((E_PAD,), N, jnp.int32).at[:E].set(
        edge_index[0].astype(jnp.int32)).reshape(R, CHUNK)
    dst = jnp.full((E_PAD,), N, jnp.int32).at[:E].set(
        edge_index[1].astype(jnp.int32)).reshape(R, CHUNK)
    w2_pad = jnp.zeros((D, HIDDEN), jnp.float32).at[:, :2].set(lin2_W)
    b2_pad = jnp.zeros((1, HIDDEN), jnp.float32).at[:, :2].set(lin2_b)

    degp = _sc_degree(src).reshape(NC, N_PAD)
    p0, p1, p2, dis = _t1(x_pad, lin1_W, lin1_b.reshape(1, HIDDEN), conv1_W, degp)

    q1 = _sc_spmm(p2, src, dst).reshape(NC, N_PAD, D)
    z1 = _tmid(p1, q1, dis)
    q2 = _sc_spmm(z1, src, dst).reshape(NC, N_PAD, D)
    r0, r1, r2 = _t3(p0, q2, dis, conv1_b.reshape(1, D), conv2_W)

    q3 = _sc_spmm(r2, src, dst).reshape(NC, N_PAD, D)
    z2 = _tmid(r1, q3, dis)
    q4 = _sc_spmm(z2, src, dst).reshape(NC, N_PAD, D)
    out = _t5(r0, q4, dis, conv2_b.reshape(1, D), w2_pad, b2_pad)

    return out[:N, :2]


# speedup vs baseline: 3.2843x; 1.3149x over previous
import functools

import jax
import jax.numpy as jnp
from jax import lax
from jax.experimental import pallas as pl
from jax.experimental.pallas import tpu as pltpu
from jax.experimental.pallas import tpu_sc as plsc

N = 10000
D_IN = 128
HIDDEN = 128
D = 64
E = 320000

NC = 2
NS = 16
N_PAD = 10240
CHUNK = 64
E_PAD = 327680
R = E_PAD // CHUNK
ROWS_SUB = N_PAD // NS
KD = 6
KG = 3
RT0 = 160
RT1 = 160
RT_MAX = max(RT0, RT1)
RT_EQ = R // (NC * NS)

_mesh = plsc.VectorSubcoreMesh(core_axis_name="c", subcore_axis_name="s")


@functools.partial(
    pl.kernel,
    mesh=_mesh,
    out_type=jax.ShapeDtypeStruct((NC * N_PAD, D), jnp.bfloat16),
    compiler_params=pltpu.CompilerParams(use_tc_tiling_on_sc=False),
    scratch_types=[
        pltpu.VMEM((RT_MAX, CHUNK), jnp.int32),
        pltpu.VMEM((RT_MAX, CHUNK), jnp.int32),
        pltpu.VMEM((KD, CHUNK, D), jnp.bfloat16),
        pltpu.SemaphoreType.DMA,
        pltpu.SemaphoreType.DMA,
        pltpu.VMEM_SHARED((N_PAD, D), jnp.bfloat16),
        pltpu.VMEM_SHARED((N_PAD, D), jnp.bfloat16),
    ],
)
def _sc_spmm(u_hbm, src_hbm, dst_hbm, out_hbm, src_v, dst_v, ring_v, gsem, ssem, acc_sh, u_sh):
    c = lax.axis_index("c")
    s = lax.axis_index("s")
    base = s * ROWS_SUB
    n_rows = jnp.where(c == 0, RT0, RT1)
    tile_base = jnp.where(c == 0, s * RT0, NS * RT0 + s * RT1)

    pltpu.sync_copy(src_hbm.at[pl.ds(tile_base, RT_MAX)], src_v)
    pltpu.sync_copy(dst_hbm.at[pl.ds(tile_base, RT_MAX)], dst_v)
    pltpu.sync_copy(u_hbm.at[pl.ds(base, ROWS_SUB)], u_sh.at[pl.ds(base, ROWS_SUB)])

    zero32 = jnp.zeros((32,), jnp.bfloat16)
    for i in range(CHUNK):
        for j in range(D // 32):
            ring_v[0, i, pl.ds(j * 32, 32)] = zero32
    for k in range(ROWS_SUB // CHUNK):
        pltpu.sync_copy(ring_v.at[0], acc_sh.at[pl.ds(base + k * CHUNK, CHUNK)])
    plsc.subcore_barrier()

    def pstep(g, _):
        for b in range(KD):
            r = g * KD + b

            @pl.when(jnp.logical_and(r >= KD, r - KD < n_rows))
            def _():
                pltpu.make_async_copy(
                    ring_v.at[b], acc_sh.at[dst_v.at[r - KD]], ssem).wait()

            @pl.when(r < n_rows)
            def _():
                pltpu.async_copy(u_sh.at[src_v.at[r]], ring_v.at[b], gsem)

            bg = (b - KG) % KD

            @pl.when(jnp.logical_and(r >= KG, r - KG < n_rows))
            def _():
                pltpu.make_async_copy(
                    u_sh.at[src_v.at[r - KG]], ring_v.at[bg], gsem).wait()
                pltpu.async_copy(
                    ring_v.at[bg], acc_sh.at[dst_v.at[r - KG]], ssem, add=True)
        return 0

    lax.fori_loop(0, RT_MAX // KD + 2, pstep, 0)
    plsc.subcore_barrier()

    def obody(k, _):
        off = base + k * CHUNK
        pltpu.sync_copy(acc_sh.at[pl.ds(off, CHUNK)], ring_v.at[0])
        pltpu.sync_copy(ring_v.at[0], out_hbm.at[pl.ds(c * N_PAD + off, CHUNK)])
        return 0

    lax.fori_loop(0, ROWS_SUB // CHUNK, obody, 0)


@functools.partial(
    pl.kernel,
    mesh=_mesh,
    out_type=jax.ShapeDtypeStruct((NC * N_PAD,), jnp.float32),
    scratch_types=[
        pltpu.VMEM((RT_EQ, CHUNK), jnp.int32),
        pltpu.VMEM((1, CHUNK), jnp.float32),
        pltpu.VMEM((ROWS_SUB,), jnp.float32),
        pltpu.SemaphoreType.DMA,
        pltpu.VMEM_SHARED((N_PAD,), jnp.float32),
    ],
)
def _sc_degree(src_hbm, out_hbm, src_v, ones_v, buf_v, ssem, acc_sh):
    c = lax.axis_index("c")
    s = lax.axis_index("s")

    one16 = jnp.ones((16,), jnp.float32)
    zero16 = jnp.zeros((16,), jnp.float32)
    for j in range(CHUNK // 16):
        ones_v[0, pl.ds(j * 16, 16)] = one16

    def zb(k, _):
        buf_v[pl.ds(k * 16, 16)] = zero16
        return 0

    lax.fori_loop(0, ROWS_SUB // 16, zb, 0)
    base = s * ROWS_SUB
    tile_base = (c * NS + s) * RT_EQ
    pltpu.sync_copy(src_hbm.at[pl.ds(tile_base, RT_EQ)], src_v)
    pltpu.sync_copy(buf_v, acc_sh.at[pl.ds(base, ROWS_SUB)])
    plsc.subcore_barrier()

    GRP = 16

    def ebody(g, _):
        for b in range(GRP):
            pltpu.async_copy(
                ones_v.at[0], acc_sh.at[src_v.at[g * GRP + b]], ssem, add=True)
        for b in range(GRP):
            pltpu.make_async_copy(
                ones_v.at[0], acc_sh.at[src_v.at[g * GRP + b]], ssem).wait()
        return 0

    lax.fori_loop(0, RT_EQ // GRP, ebody, 0)
    plsc.subcore_barrier()

    pltpu.sync_copy(acc_sh.at[pl.ds(base, ROWS_SUB)], buf_v)
    pltpu.sync_copy(buf_v, out_hbm.at[pl.ds(c * N_PAD + base, ROWS_SUB)])


def _t1_body(x_ref, w1_ref, b1_ref, cw_ref, degp_ref, p0_ref, p1_ref, p2_ref, dis_ref):
    degp = degp_ref[...]
    deg = jnp.reshape(degp[0] + degp[1], (N_PAD, 1))
    rid = lax.broadcasted_iota(jnp.int32, (N_PAD, 1), 0)
    dis = jnp.where((rid < N) & (deg > 0.0),
                    lax.rsqrt(jnp.maximum(deg, 1e-12)), 0.0)
    dis_ref[...] = dis
    h = jnp.maximum(
        jnp.dot(x_ref[...], w1_ref[...], preferred_element_type=jnp.float32)
        + b1_ref[...], 0.0)
    cw = cw_ref[...]
    p0_ref[...] = jnp.dot(h, cw[0] - cw[2], preferred_element_type=jnp.float32)
    p1_ref[...] = jnp.dot(h, cw[1], preferred_element_type=jnp.float32)
    p2_ref[...] = (dis * jnp.dot(h, cw[2], preferred_element_type=jnp.float32)).astype(jnp.bfloat16)


_t1 = pl.pallas_call(
    _t1_body,
    out_shape=[
        jax.ShapeDtypeStruct((N_PAD, D), jnp.float32),
        jax.ShapeDtypeStruct((N_PAD, D), jnp.float32),
        jax.ShapeDtypeStruct((N_PAD, D), jnp.bfloat16),
        jax.ShapeDtypeStruct((N_PAD, 1), jnp.float32),
    ],
)


def _tmid_body(p1_ref, q_ref, dis_ref, z_ref):
    q = q_ref[...].astype(jnp.float32)
    dis = dis_ref[...]
    z_ref[...] = (dis * p1_ref[...]
                  - 2.0 * dis * dis * (q[0] + q[1])).astype(jnp.bfloat16)


_tmid = pl.pallas_call(
    _tmid_body,
    out_shape=jax.ShapeDtypeStruct((N_PAD, D), jnp.bfloat16),
)


def _t3_body(p0_ref, q_ref, dis_ref, bc_ref, cw_ref, r0_ref, r1_ref, r2_ref):
    q = q_ref[...].astype(jnp.float32)
    dis = dis_ref[...]
    g = jnp.maximum(p0_ref[...] - dis * (q[0] + q[1]) + bc_ref[...], 0.0)
    cw = cw_ref[...]
    r0_ref[...] = jnp.dot(g, cw[0] - cw[2], preferred_element_type=jnp.float32)
    r1_ref[...] = jnp.dot(g, cw[1], preferred_element_type=jnp.float32)
    r2_ref[...] = (dis * jnp.dot(g, cw[2], preferred_element_type=jnp.float32)).astype(jnp.bfloat16)


_t3 = pl.pallas_call(
    _t3_body,
    out_shape=[
        jax.ShapeDtypeStruct((N_PAD, D), jnp.float32),
        jax.ShapeDtypeStruct((N_PAD, D), jnp.float32),
        jax.ShapeDtypeStruct((N_PAD, D), jnp.bfloat16),
    ],
)


def _t5_body(r0_ref, q_ref, dis_ref, bc_ref, w2_ref, b2_ref, out_ref):
    q = q_ref[...].astype(jnp.float32)
    dis = dis_ref[...]
    f = jnp.maximum(r0_ref[...] - dis * (q[0] + q[1]) + bc_ref[...], 0.0)
    logits = jnp.dot(f, w2_ref[...], preferred_element_type=jnp.float32) + b2_ref[...]
    col = lax.broadcasted_iota(jnp.int32, (N_PAD, HIDDEN), 1)
    mask = col < 2
    ml = jnp.where(mask, logits, -jnp.inf)
    m = jnp.max(ml, axis=1, keepdims=True)
    e = jnp.where(mask, jnp.exp(logits - m), 0.0)
    out_ref[...] = e / jnp.sum(e, axis=1, keepdims=True)


_t5 = pl.pallas_call(
    _t5_body,
    out_shape=jax.ShapeDtypeStruct((N_PAD, HIDDEN), jnp.float32),
)


@jax.jit
def kernel(x, edge_index, lin1_W, lin1_b, conv1_W, conv1_b, conv2_W, conv2_b,
           lin2_W, lin2_b):
    x_pad = jnp.zeros((N_PAD, D_IN), jnp.float32).at[:N].set(x)
    src = jnp.full((E_PAD,), N, jnp.int32).at[:E].set(
        edge_index[0].astype(jnp.int32)).reshape(R, CHUNK)
    dst = jnp.full((E_PAD,), N, jnp.int32).at[:E].set(
        edge_index[1].astype(jnp.int32)).reshape(R, CHUNK)
    w2_pad = jnp.zeros((D, HIDDEN), jnp.float32).at[:, :2].set(lin2_W)
    b2_pad = jnp.zeros((1, HIDDEN), jnp.float32).at[:, :2].set(lin2_b)

    degp = _sc_degree(src).reshape(NC, N_PAD)
    p0, p1, p2, dis = _t1(x_pad, lin1_W, lin1_b.reshape(1, HIDDEN), conv1_W, degp)

    q1 = _sc_spmm(p2, src, dst).reshape(NC, N_PAD, D)
    z1 = _tmid(p1, q1, dis)
    q2 = _sc_spmm(z1, src, dst).reshape(NC, N_PAD, D)
    r0, r1, r2 = _t3(p0, q2, dis, conv1_b.reshape(1, D), conv2_W)

    q3 = _sc_spmm(r2, src, dst).reshape(NC, N_PAD, D)
    z2 = _tmid(r1, q3, dis)
    q4 = _sc_spmm(z2, src, dst).reshape(NC, N_PAD, D)
    out = _t5(r0, q4, dis, conv2_b.reshape(1, D), w2_pad, b2_pad)

    return out[:N, :2]


# speedup vs baseline: 3.3236x; 1.0120x over previous
import functools

import jax
import jax.numpy as jnp
from jax import lax
from jax.experimental import pallas as pl
from jax.experimental.pallas import tpu as pltpu
from jax.experimental.pallas import tpu_sc as plsc

N = 10000
D_IN = 128
HIDDEN = 128
D = 64
E = 320000

NC = 2
NS = 16
N_PAD = 10240
CHUNK = 128
E_PAD = 327680
R = E_PAD // CHUNK
ROWS_SUB = N_PAD // NS
KD = 6
KG = 3
RT0 = 80
RT1 = 80
RT_MAX = max(RT0, RT1)
RT_EQ = R // (NC * NS)

_mesh = plsc.VectorSubcoreMesh(core_axis_name="c", subcore_axis_name="s")


@functools.partial(
    pl.kernel,
    mesh=_mesh,
    out_type=jax.ShapeDtypeStruct((NC * N_PAD, D), jnp.bfloat16),
    compiler_params=pltpu.CompilerParams(use_tc_tiling_on_sc=False),
    scratch_types=[
        pltpu.VMEM((RT_MAX, CHUNK), jnp.int32),
        pltpu.VMEM((RT_MAX, CHUNK), jnp.int32),
        pltpu.VMEM((KD, CHUNK, D), jnp.bfloat16),
        pltpu.SemaphoreType.DMA,
        pltpu.SemaphoreType.DMA,
        pltpu.VMEM_SHARED((N_PAD, D), jnp.bfloat16),
        pltpu.VMEM_SHARED((N_PAD, D), jnp.bfloat16),
    ],
)
def _sc_spmm(u_hbm, src_hbm, dst_hbm, out_hbm, src_v, dst_v, ring_v, gsem, ssem, acc_sh, u_sh):
    c = lax.axis_index("c")
    s = lax.axis_index("s")
    base = s * ROWS_SUB
    n_rows = jnp.where(c == 0, RT0, RT1)
    tile_base = jnp.where(c == 0, s * RT0, NS * RT0 + s * RT1)

    pltpu.sync_copy(src_hbm.at[pl.ds(tile_base, RT_MAX)], src_v)
    pltpu.sync_copy(dst_hbm.at[pl.ds(tile_base, RT_MAX)], dst_v)
    pltpu.sync_copy(u_hbm.at[pl.ds(base, ROWS_SUB)], u_sh.at[pl.ds(base, ROWS_SUB)])

    zero32 = jnp.zeros((32,), jnp.bfloat16)
    for i in range(CHUNK):
        for j in range(D // 32):
            ring_v[0, i, pl.ds(j * 32, 32)] = zero32
    for k in range(ROWS_SUB // CHUNK):
        pltpu.sync_copy(ring_v.at[0], acc_sh.at[pl.ds(base + k * CHUNK, CHUNK)])
    plsc.subcore_barrier()

    def pstep(g, _):
        for b in range(KD):
            r = g * KD + b

            @pl.when(jnp.logical_and(r >= KD, r - KD < n_rows))
            def _():
                pltpu.make_async_copy(
                    ring_v.at[b], acc_sh.at[dst_v.at[r - KD]], ssem).wait()

            @pl.when(r < n_rows)
            def _():
                pltpu.async_copy(u_sh.at[src_v.at[r]], ring_v.at[b], gsem)

            bg = (b - KG) % KD

            @pl.when(jnp.logical_and(r >= KG, r - KG < n_rows))
            def _():
                pltpu.make_async_copy(
                    u_sh.at[src_v.at[r - KG]], ring_v.at[bg], gsem).wait()
                pltpu.async_copy(
                    ring_v.at[bg], acc_sh.at[dst_v.at[r - KG]], ssem, add=True)
        return 0

    lax.fori_loop(0, RT_MAX // KD + 2, pstep, 0)
    plsc.subcore_barrier()

    def obody(k, _):
        off = base + k * CHUNK
        pltpu.sync_copy(acc_sh.at[pl.ds(off, CHUNK)], ring_v.at[0])
        pltpu.sync_copy(ring_v.at[0], out_hbm.at[pl.ds(c * N_PAD + off, CHUNK)])
        return 0

    lax.fori_loop(0, ROWS_SUB // CHUNK, obody, 0)


@functools.partial(
    pl.kernel,
    mesh=_mesh,
    out_type=jax.ShapeDtypeStruct((NC * N_PAD,), jnp.float32),
    scratch_types=[
        pltpu.VMEM((RT_EQ, CHUNK), jnp.int32),
        pltpu.VMEM((1, CHUNK), jnp.float32),
        pltpu.VMEM((ROWS_SUB,), jnp.float32),
        pltpu.SemaphoreType.DMA,
        pltpu.VMEM_SHARED((N_PAD,), jnp.float32),
    ],
)
def _sc_degree(src_hbm, out_hbm, src_v, ones_v, buf_v, ssem, acc_sh):
    c = lax.axis_index("c")
    s = lax.axis_index("s")

    one16 = jnp.ones((16,), jnp.float32)
    zero16 = jnp.zeros((16,), jnp.float32)
    for j in range(CHUNK // 16):
        ones_v[0, pl.ds(j * 16, 16)] = one16

    def zb(k, _):
        buf_v[pl.ds(k * 16, 16)] = zero16
        return 0

    lax.fori_loop(0, ROWS_SUB // 16, zb, 0)
    base = s * ROWS_SUB
    tile_base = (c * NS + s) * RT_EQ
    pltpu.sync_copy(src_hbm.at[pl.ds(tile_base, RT_EQ)], src_v)
    pltpu.sync_copy(buf_v, acc_sh.at[pl.ds(base, ROWS_SUB)])
    plsc.subcore_barrier()

    GRP = 16

    def ebody(g, _):
        for b in range(GRP):
            pltpu.async_copy(
                ones_v.at[0], acc_sh.at[src_v.at[g * GRP + b]], ssem, add=True)
        for b in range(GRP):
            pltpu.make_async_copy(
                ones_v.at[0], acc_sh.at[src_v.at[g * GRP + b]], ssem).wait()
        return 0

    lax.fori_loop(0, RT_EQ // GRP, ebody, 0)
    plsc.subcore_barrier()

    pltpu.sync_copy(acc_sh.at[pl.ds(base, ROWS_SUB)], buf_v)
    pltpu.sync_copy(buf_v, out_hbm.at[pl.ds(c * N_PAD + base, ROWS_SUB)])


def _t1_body(x_ref, w1_ref, b1_ref, cw_ref, degp_ref, p0_ref, p1_ref, p2_ref, dis_ref):
    degp = degp_ref[...]
    deg = jnp.reshape(degp[0] + degp[1], (N_PAD, 1))
    rid = lax.broadcasted_iota(jnp.int32, (N_PAD, 1), 0)
    dis = jnp.where((rid < N) & (deg > 0.0),
                    lax.rsqrt(jnp.maximum(deg, 1e-12)), 0.0)
    dis_ref[...] = dis
    h = jnp.maximum(
        jnp.dot(x_ref[...], w1_ref[...], preferred_element_type=jnp.float32)
        + b1_ref[...], 0.0)
    cw = cw_ref[...]
    p0_ref[...] = jnp.dot(h, cw[0] - cw[2], preferred_element_type=jnp.float32)
    p1_ref[...] = jnp.dot(h, cw[1], preferred_element_type=jnp.float32)
    p2_ref[...] = (dis * jnp.dot(h, cw[2], preferred_element_type=jnp.float32)).astype(jnp.bfloat16)


_t1 = pl.pallas_call(
    _t1_body,
    out_shape=[
        jax.ShapeDtypeStruct((N_PAD, D), jnp.float32),
        jax.ShapeDtypeStruct((N_PAD, D), jnp.float32),
        jax.ShapeDtypeStruct((N_PAD, D), jnp.bfloat16),
        jax.ShapeDtypeStruct((N_PAD, 1), jnp.float32),
    ],
)


def _tmid_body(p1_ref, q_ref, dis_ref, z_ref):
    q = q_ref[...].astype(jnp.float32)
    dis = dis_ref[...]
    z_ref[...] = (dis * p1_ref[...]
                  - 2.0 * dis * dis * (q[0] + q[1])).astype(jnp.bfloat16)


_tmid = pl.pallas_call(
    _tmid_body,
    out_shape=jax.ShapeDtypeStruct((N_PAD, D), jnp.bfloat16),
)


def _t3_body(p0_ref, q_ref, dis_ref, bc_ref, cw_ref, r0_ref, r1_ref, r2_ref):
    q = q_ref[...].astype(jnp.float32)
    dis = dis_ref[...]
    g = jnp.maximum(p0_ref[...] - dis * (q[0] + q[1]) + bc_ref[...], 0.0)
    cw = cw_ref[...]
    r0_ref[...] = jnp.dot(g, cw[0] - cw[2], preferred_element_type=jnp.float32)
    r1_ref[...] = jnp.dot(g, cw[1], preferred_element_type=jnp.float32)
    r2_ref[...] = (dis * jnp.dot(g, cw[2], preferred_element_type=jnp.float32)).astype(jnp.bfloat16)


_t3 = pl.pallas_call(
    _t3_body,
    out_shape=[
        jax.ShapeDtypeStruct((N_PAD, D), jnp.float32),
        jax.ShapeDtypeStruct((N_PAD, D), jnp.float32),
        jax.ShapeDtypeStruct((N_PAD, D), jnp.bfloat16),
    ],
)


def _t5_body(r0_ref, q_ref, dis_ref, bc_ref, w2_ref, b2_ref, out_ref):
    q = q_ref[...].astype(jnp.float32)
    dis = dis_ref[...]
    f = jnp.maximum(r0_ref[...] - dis * (q[0] + q[1]) + bc_ref[...], 0.0)
    logits = jnp.dot(f, w2_ref[...], preferred_element_type=jnp.float32) + b2_ref[...]
    col = lax.broadcasted_iota(jnp.int32, (N_PAD, HIDDEN), 1)
    mask = col < 2
    ml = jnp.where(mask, logits, -jnp.inf)
    m = jnp.max(ml, axis=1, keepdims=True)
    e = jnp.where(mask, jnp.exp(logits - m), 0.0)
    out_ref[...] = e / jnp.sum(e, axis=1, keepdims=True)


_t5 = pl.pallas_call(
    _t5_body,
    out_shape=jax.ShapeDtypeStruct((N_PAD, HIDDEN), jnp.float32),
)


@jax.jit
def kernel(x, edge_index, lin1_W, lin1_b, conv1_W, conv1_b, conv2_W, conv2_b,
           lin2_W, lin2_b):
    x_pad = jnp.zeros((N_PAD, D_IN), jnp.float32).at[:N].set(x)
    src = jnp.full((E_PAD,), N, jnp.int32).at[:E].set(
        edge_index[0].astype(jnp.int32)).reshape(R, CHUNK)
    dst = jnp.full((E_PAD,), N, jnp.int32).at[:E].set(
        edge_index[1].astype(jnp.int32)).reshape(R, CHUNK)
    w2_pad = jnp.zeros((D, HIDDEN), jnp.float32).at[:, :2].set(lin2_W)
    b2_pad = jnp.zeros((1, HIDDEN), jnp.float32).at[:, :2].set(lin2_b)

    degp = _sc_degree(src).reshape(NC, N_PAD)
    p0, p1, p2, dis = _t1(x_pad, lin1_W, lin1_b.reshape(1, HIDDEN), conv1_W, degp)

    q1 = _sc_spmm(p2, src, dst).reshape(NC, N_PAD, D)
    z1 = _tmid(p1, q1, dis)
    q2 = _sc_spmm(z1, src, dst).reshape(NC, N_PAD, D)
    r0, r1, r2 = _t3(p0, q2, dis, conv1_b.reshape(1, D), conv2_W)

    q3 = _sc_spmm(r2, src, dst).reshape(NC, N_PAD, D)
    z2 = _tmid(r1, q3, dis)
    q4 = _sc_spmm(z2, src, dst).reshape(NC, N_PAD, D)
    out = _t5(r0, q4, dis, conv2_b.reshape(1, D), w2_pad, b2_pad)

    return out[:N, :2]


# speedup vs baseline: 3.4598x; 1.0410x over previous
import functools

import jax
import jax.numpy as jnp
from jax import lax
from jax.experimental import pallas as pl
from jax.experimental.pallas import tpu as pltpu
from jax.experimental.pallas import tpu_sc as plsc

N = 10000
D_IN = 128
HIDDEN = 128
D = 64
E = 320000

NC = 2
NS = 16
N_PAD = 10240
CHUNK = 128
E_PAD = 327680
R = E_PAD // CHUNK
ROWS_SUB = N_PAD // NS
KD = 6
KG = 3
RT0 = 80
RT1 = 80
RT_MAX = max(RT0, RT1)
RT_EQ = R // (NC * NS)

_mesh = plsc.VectorSubcoreMesh(core_axis_name="c", subcore_axis_name="s")


@functools.partial(
    pl.kernel,
    mesh=_mesh,
    out_type=jax.ShapeDtypeStruct((NC * N_PAD, D), jnp.bfloat16),
    compiler_params=pltpu.CompilerParams(use_tc_tiling_on_sc=False),
    scratch_types=[
        pltpu.VMEM((RT_MAX, CHUNK), jnp.int32),
        pltpu.VMEM((RT_MAX, CHUNK), jnp.int32),
        pltpu.VMEM((KD, CHUNK, D), jnp.bfloat16),
        pltpu.SemaphoreType.DMA,
        pltpu.SemaphoreType.DMA,
        pltpu.VMEM_SHARED((N_PAD, D), jnp.bfloat16),
        pltpu.VMEM_SHARED((N_PAD, D), jnp.bfloat16),
    ],
)
def _sc_spmm(u_hbm, src_hbm, dst_hbm, out_hbm, src_v, dst_v, ring_v, gsem, ssem, acc_sh, u_sh):
    c = lax.axis_index("c")
    s = lax.axis_index("s")
    base = s * ROWS_SUB
    n_rows = jnp.where(c == 0, RT0, RT1)
    tile_base = jnp.where(c == 0, s * RT0, NS * RT0 + s * RT1)

    zero32 = jnp.zeros((32,), jnp.bfloat16)
    for i in range(CHUNK):
        for j in range(D // 32):
            ring_v[0, i, pl.ds(j * 32, 32)] = zero32
    pltpu.async_copy(src_hbm.at[pl.ds(tile_base, RT_MAX)], src_v, gsem)
    pltpu.async_copy(dst_hbm.at[pl.ds(tile_base, RT_MAX)], dst_v, gsem)
    pltpu.async_copy(u_hbm.at[pl.ds(base, ROWS_SUB)],
                     u_sh.at[pl.ds(base, ROWS_SUB)], gsem)
    for k in range(ROWS_SUB // CHUNK):
        pltpu.async_copy(ring_v.at[0], acc_sh.at[pl.ds(base + k * CHUNK, CHUNK)],
                         ssem)
    pltpu.make_async_copy(src_hbm.at[pl.ds(tile_base, RT_MAX)], src_v, gsem).wait()
    pltpu.make_async_copy(dst_hbm.at[pl.ds(tile_base, RT_MAX)], dst_v, gsem).wait()
    pltpu.make_async_copy(u_hbm.at[pl.ds(base, ROWS_SUB)],
                          u_sh.at[pl.ds(base, ROWS_SUB)], gsem).wait()
    for k in range(ROWS_SUB // CHUNK):
        pltpu.make_async_copy(ring_v.at[0],
                              acc_sh.at[pl.ds(base + k * CHUNK, CHUNK)],
                              ssem).wait()
    plsc.subcore_barrier()

    def pstep(g, _):
        for b in range(KD):
            r = g * KD + b

            @pl.when(jnp.logical_and(r >= KD, r - KD < n_rows))
            def _():
                pltpu.make_async_copy(
                    ring_v.at[b], acc_sh.at[dst_v.at[r - KD]], ssem).wait()

            @pl.when(r < n_rows)
            def _():
                pltpu.async_copy(u_sh.at[src_v.at[r]], ring_v.at[b], gsem)

            bg = (b - KG) % KD

            @pl.when(jnp.logical_and(r >= KG, r - KG < n_rows))
            def _():
                pltpu.make_async_copy(
                    u_sh.at[src_v.at[r - KG]], ring_v.at[bg], gsem).wait()
                pltpu.async_copy(
                    ring_v.at[bg], acc_sh.at[dst_v.at[r - KG]], ssem, add=True)
        return 0

    lax.fori_loop(0, RT_MAX // KD + 2, pstep, 0)
    plsc.subcore_barrier()

    NOUT = ROWS_SUB // CHUNK
    for k in range(NOUT):
        pltpu.async_copy(acc_sh.at[pl.ds(base + k * CHUNK, CHUNK)],
                         ring_v.at[k % 2], gsem)
        if k:
            pltpu.make_async_copy(
                ring_v.at[(k - 1) % 2],
                out_hbm.at[pl.ds(c * N_PAD + base + (k - 1) * CHUNK, CHUNK)],
                ssem).wait()
        pltpu.make_async_copy(acc_sh.at[pl.ds(base + k * CHUNK, CHUNK)],
                              ring_v.at[k % 2], gsem).wait()
        pltpu.async_copy(ring_v.at[k % 2],
                         out_hbm.at[pl.ds(c * N_PAD + base + k * CHUNK, CHUNK)],
                         ssem)
    pltpu.make_async_copy(
        ring_v.at[(NOUT - 1) % 2],
        out_hbm.at[pl.ds(c * N_PAD + base + (NOUT - 1) * CHUNK, CHUNK)],
        ssem).wait()


@functools.partial(
    pl.kernel,
    mesh=_mesh,
    out_type=jax.ShapeDtypeStruct((NC * N_PAD,), jnp.float32),
    scratch_types=[
        pltpu.VMEM((RT_EQ, CHUNK), jnp.int32),
        pltpu.VMEM((1, CHUNK), jnp.float32),
        pltpu.VMEM((ROWS_SUB,), jnp.float32),
        pltpu.SemaphoreType.DMA,
        pltpu.VMEM_SHARED((N_PAD,), jnp.float32),
    ],
)
def _sc_degree(src_hbm, out_hbm, src_v, ones_v, buf_v, ssem, acc_sh):
    c = lax.axis_index("c")
    s = lax.axis_index("s")

    one16 = jnp.ones((16,), jnp.float32)
    zero16 = jnp.zeros((16,), jnp.float32)
    for j in range(CHUNK // 16):
        ones_v[0, pl.ds(j * 16, 16)] = one16

    def zb(k, _):
        buf_v[pl.ds(k * 16, 16)] = zero16
        return 0

    lax.fori_loop(0, ROWS_SUB // 16, zb, 0)
    base = s * ROWS_SUB
    tile_base = (c * NS + s) * RT_EQ
    pltpu.sync_copy(src_hbm.at[pl.ds(tile_base, RT_EQ)], src_v)
    pltpu.sync_copy(buf_v, acc_sh.at[pl.ds(base, ROWS_SUB)])
    plsc.subcore_barrier()

    GRP = 16

    def ebody(g, _):
        for b in range(GRP):
            pltpu.async_copy(
                ones_v.at[0], acc_sh.at[src_v.at[g * GRP + b]], ssem, add=True)
        for b in range(GRP):
            pltpu.make_async_copy(
                ones_v.at[0], acc_sh.at[src_v.at[g * GRP + b]], ssem).wait()
        return 0

    lax.fori_loop(0, RT_EQ // GRP, ebody, 0)
    plsc.subcore_barrier()

    pltpu.sync_copy(acc_sh.at[pl.ds(base, ROWS_SUB)], buf_v)
    pltpu.sync_copy(buf_v, out_hbm.at[pl.ds(c * N_PAD + base, ROWS_SUB)])


def _t1_body(x_ref, w1_ref, b1_ref, cw_ref, degp_ref, p0_ref, p1_ref, p2_ref, dis_ref):
    degp = degp_ref[...]
    deg = jnp.reshape(degp[0] + degp[1], (N_PAD, 1))
    rid = lax.broadcasted_iota(jnp.int32, (N_PAD, 1), 0)
    dis = jnp.where((rid < N) & (deg > 0.0),
                    lax.rsqrt(jnp.maximum(deg, 1e-12)), 0.0)
    dis_ref[...] = dis
    h = jnp.maximum(
        jnp.dot(x_ref[...], w1_ref[...], preferred_element_type=jnp.float32)
        + b1_ref[...], 0.0)
    cw = cw_ref[...]
    p0_ref[...] = jnp.dot(h, cw[0] - cw[2], preferred_element_type=jnp.float32)
    p1_ref[...] = jnp.dot(h, cw[1], preferred_element_type=jnp.float32)
    p2_ref[...] = (dis * jnp.dot(h, cw[2], preferred_element_type=jnp.float32)).astype(jnp.bfloat16)


_t1 = pl.pallas_call(
    _t1_body,
    out_shape=[
        jax.ShapeDtypeStruct((N_PAD, D), jnp.float32),
        jax.ShapeDtypeStruct((N_PAD, D), jnp.float32),
        jax.ShapeDtypeStruct((N_PAD, D), jnp.bfloat16),
        jax.ShapeDtypeStruct((N_PAD, 1), jnp.float32),
    ],
)


def _tmid_body(p1_ref, q_ref, dis_ref, z_ref):
    q = q_ref[...].astype(jnp.float32)
    dis = dis_ref[...]
    z_ref[...] = (dis * p1_ref[...]
                  - 2.0 * dis * dis * (q[0] + q[1])).astype(jnp.bfloat16)


_tmid = pl.pallas_call(
    _tmid_body,
    out_shape=jax.ShapeDtypeStruct((N_PAD, D), jnp.bfloat16),
)


def _t3_body(p0_ref, q_ref, dis_ref, bc_ref, cw_ref, r0_ref, r1_ref, r2_ref):
    q = q_ref[...].astype(jnp.float32)
    dis = dis_ref[...]
    g = jnp.maximum(p0_ref[...] - dis * (q[0] + q[1]) + bc_ref[...], 0.0)
    cw = cw_ref[...]
    r0_ref[...] = jnp.dot(g, cw[0] - cw[2], preferred_element_type=jnp.float32)
    r1_ref[...] = jnp.dot(g, cw[1], preferred_element_type=jnp.float32)
    r2_ref[...] = (dis * jnp.dot(g, cw[2], preferred_element_type=jnp.float32)).astype(jnp.bfloat16)


_t3 = pl.pallas_call(
    _t3_body,
    out_shape=[
        jax.ShapeDtypeStruct((N_PAD, D), jnp.float32),
        jax.ShapeDtypeStruct((N_PAD, D), jnp.float32),
        jax.ShapeDtypeStruct((N_PAD, D), jnp.bfloat16),
    ],
)


def _t5_body(r0_ref, q_ref, dis_ref, bc_ref, w2_ref, b2_ref, out_ref):
    q = q_ref[...].astype(jnp.float32)
    dis = dis_ref[...]
    f = jnp.maximum(r0_ref[...] - dis * (q[0] + q[1]) + bc_ref[...], 0.0)
    logits = jnp.dot(f, w2_ref[...], preferred_element_type=jnp.float32) + b2_ref[...]
    col = lax.broadcasted_iota(jnp.int32, (N_PAD, HIDDEN), 1)
    mask = col < 2
    ml = jnp.where(mask, logits, -jnp.inf)
    m = jnp.max(ml, axis=1, keepdims=True)
    e = jnp.where(mask, jnp.exp(logits - m), 0.0)
    out_ref[...] = e / jnp.sum(e, axis=1, keepdims=True)


_t5 = pl.pallas_call(
    _t5_body,
    out_shape=jax.ShapeDtypeStruct((N_PAD, HIDDEN), jnp.float32),
)


@jax.jit
def kernel(x, edge_index, lin1_W, lin1_b, conv1_W, conv1_b, conv2_W, conv2_b,
           lin2_W, lin2_b):
    x_pad = jnp.zeros((N_PAD, D_IN), jnp.float32).at[:N].set(x)
    src = jnp.full((E_PAD,), N, jnp.int32).at[:E].set(
        edge_index[0].astype(jnp.int32)).reshape(R, CHUNK)
    dst = jnp.full((E_PAD,), N, jnp.int32).at[:E].set(
        edge_index[1].astype(jnp.int32)).reshape(R, CHUNK)
    w2_pad = jnp.zeros((D, HIDDEN), jnp.float32).at[:, :2].set(lin2_W)
    b2_pad = jnp.zeros((1, HIDDEN), jnp.float32).at[:, :2].set(lin2_b)

    degp = _sc_degree(src).reshape(NC, N_PAD)
    p0, p1, p2, dis = _t1(x_pad, lin1_W, lin1_b.reshape(1, HIDDEN), conv1_W, degp)

    q1 = _sc_spmm(p2, src, dst).reshape(NC, N_PAD, D)
    z1 = _tmid(p1, q1, dis)
    q2 = _sc_spmm(z1, src, dst).reshape(NC, N_PAD, D)
    r0, r1, r2 = _t3(p0, q2, dis, conv1_b.reshape(1, D), conv2_W)

    q3 = _sc_spmm(r2, src, dst).reshape(NC, N_PAD, D)
    z2 = _tmid(r1, q3, dis)
    q4 = _sc_spmm(z2, src, dst).reshape(NC, N_PAD, D)
    out = _t5(r0, q4, dis, conv2_b.reshape(1, D), w2_pad, b2_pad)

    return out[:N, :2]


# speedup vs baseline: 3.4600x; 1.0001x over previous
import functools

import jax
import jax.numpy as jnp
from jax import lax
from jax.experimental import pallas as pl
from jax.experimental.pallas import tpu as pltpu
from jax.experimental.pallas import tpu_sc as plsc

N = 10000
D_IN = 128
HIDDEN = 128
D = 64
E = 320000

NC = 2
NS = 16
N_PAD = 10240
CHUNK = 128
E_PAD = 327680
R = E_PAD // CHUNK
ROWS_SUB = N_PAD // NS
KD = 8
KG = 4
RT0 = 80
RT1 = 80
RT_MAX = max(RT0, RT1)
RT_EQ = R // (NC * NS)

_mesh = plsc.VectorSubcoreMesh(core_axis_name="c", subcore_axis_name="s")


@functools.partial(
    pl.kernel,
    mesh=_mesh,
    out_type=jax.ShapeDtypeStruct((NC * N_PAD, D), jnp.bfloat16),
    compiler_params=pltpu.CompilerParams(use_tc_tiling_on_sc=False),
    scratch_types=[
        pltpu.VMEM((RT_MAX, CHUNK), jnp.int32),
        pltpu.VMEM((RT_MAX, CHUNK), jnp.int32),
        pltpu.VMEM((KD, CHUNK, D), jnp.bfloat16),
        pltpu.SemaphoreType.DMA,
        pltpu.SemaphoreType.DMA,
        pltpu.VMEM_SHARED((N_PAD, D), jnp.bfloat16),
        pltpu.VMEM_SHARED((N_PAD, D), jnp.bfloat16),
    ],
)
def _sc_spmm(u_hbm, src_hbm, dst_hbm, out_hbm, src_v, dst_v, ring_v, gsem, ssem, acc_sh, u_sh):
    c = lax.axis_index("c")
    s = lax.axis_index("s")
    base = s * ROWS_SUB
    n_rows = jnp.where(c == 0, RT0, RT1)
    tile_base = jnp.where(c == 0, s * RT0, NS * RT0 + s * RT1)

    zero32 = jnp.zeros((32,), jnp.bfloat16)
    for i in range(CHUNK):
        for j in range(D // 32):
            ring_v[0, i, pl.ds(j * 32, 32)] = zero32
    pltpu.async_copy(src_hbm.at[pl.ds(tile_base, RT_MAX)], src_v, gsem)
    pltpu.async_copy(dst_hbm.at[pl.ds(tile_base, RT_MAX)], dst_v, gsem)
    pltpu.async_copy(u_hbm.at[pl.ds(base, ROWS_SUB)],
                     u_sh.at[pl.ds(base, ROWS_SUB)], gsem)
    for k in range(ROWS_SUB // CHUNK):
        pltpu.async_copy(ring_v.at[0], acc_sh.at[pl.ds(base + k * CHUNK, CHUNK)],
                         ssem)
    pltpu.make_async_copy(src_hbm.at[pl.ds(tile_base, RT_MAX)], src_v, gsem).wait()
    pltpu.make_async_copy(dst_hbm.at[pl.ds(tile_base, RT_MAX)], dst_v, gsem).wait()
    pltpu.make_async_copy(u_hbm.at[pl.ds(base, ROWS_SUB)],
                          u_sh.at[pl.ds(base, ROWS_SUB)], gsem).wait()
    for k in range(ROWS_SUB // CHUNK):
        pltpu.make_async_copy(ring_v.at[0],
                              acc_sh.at[pl.ds(base + k * CHUNK, CHUNK)],
                              ssem).wait()
    plsc.subcore_barrier()

    def pstep(g, _):
        for b in range(KD):
            r = g * KD + b

            @pl.when(jnp.logical_and(r >= KD, r - KD < n_rows))
            def _():
                pltpu.make_async_copy(
                    ring_v.at[b], acc_sh.at[dst_v.at[r - KD]], ssem).wait()

            @pl.when(r < n_rows)
            def _():
                pltpu.async_copy(u_sh.at[src_v.at[r]], ring_v.at[b], gsem)

            bg = (b - KG) % KD

            @pl.when(jnp.logical_and(r >= KG, r - KG < n_rows))
            def _():
                pltpu.make_async_copy(
                    u_sh.at[src_v.at[r - KG]], ring_v.at[bg], gsem).wait()
                pltpu.async_copy(
                    ring_v.at[bg], acc_sh.at[dst_v.at[r - KG]], ssem, add=True)
        return 0

    lax.fori_loop(0, RT_MAX // KD + 2, pstep, 0)
    plsc.subcore_barrier()

    NOUT = ROWS_SUB // CHUNK
    for k in range(NOUT):
        pltpu.async_copy(acc_sh.at[pl.ds(base + k * CHUNK, CHUNK)],
                         ring_v.at[k % 2], gsem)
        if k:
            pltpu.make_async_copy(
                ring_v.at[(k - 1) % 2],
                out_hbm.at[pl.ds(c * N_PAD + base + (k - 1) * CHUNK, CHUNK)],
                ssem).wait()
        pltpu.make_async_copy(acc_sh.at[pl.ds(base + k * CHUNK, CHUNK)],
                              ring_v.at[k % 2], gsem).wait()
        pltpu.async_copy(ring_v.at[k % 2],
                         out_hbm.at[pl.ds(c * N_PAD + base + k * CHUNK, CHUNK)],
                         ssem)
    pltpu.make_async_copy(
        ring_v.at[(NOUT - 1) % 2],
        out_hbm.at[pl.ds(c * N_PAD + base + (NOUT - 1) * CHUNK, CHUNK)],
        ssem).wait()


@functools.partial(
    pl.kernel,
    mesh=_mesh,
    out_type=jax.ShapeDtypeStruct((NC * N_PAD,), jnp.float32),
    scratch_types=[
        pltpu.VMEM((RT_EQ, CHUNK), jnp.int32),
        pltpu.VMEM((1, CHUNK), jnp.float32),
        pltpu.VMEM((ROWS_SUB,), jnp.float32),
        pltpu.SemaphoreType.DMA,
        pltpu.VMEM_SHARED((N_PAD,), jnp.float32),
    ],
)
def _sc_degree(src_hbm, out_hbm, src_v, ones_v, buf_v, ssem, acc_sh):
    c = lax.axis_index("c")
    s = lax.axis_index("s")

    one16 = jnp.ones((16,), jnp.float32)
    zero16 = jnp.zeros((16,), jnp.float32)
    for j in range(CHUNK // 16):
        ones_v[0, pl.ds(j * 16, 16)] = one16

    def zb(k, _):
        buf_v[pl.ds(k * 16, 16)] = zero16
        return 0

    lax.fori_loop(0, ROWS_SUB // 16, zb, 0)
    base = s * ROWS_SUB
    tile_base = (c * NS + s) * RT_EQ
    pltpu.sync_copy(src_hbm.at[pl.ds(tile_base, RT_EQ)], src_v)
    pltpu.sync_copy(buf_v, acc_sh.at[pl.ds(base, ROWS_SUB)])
    plsc.subcore_barrier()

    GRP = 16

    def ebody(g, _):
        for b in range(GRP):
            pltpu.async_copy(
                ones_v.at[0], acc_sh.at[src_v.at[g * GRP + b]], ssem, add=True)
        for b in range(GRP):
            pltpu.make_async_copy(
                ones_v.at[0], acc_sh.at[src_v.at[g * GRP + b]], ssem).wait()
        return 0

    lax.fori_loop(0, RT_EQ // GRP, ebody, 0)
    plsc.subcore_barrier()

    pltpu.sync_copy(acc_sh.at[pl.ds(base, ROWS_SUB)], buf_v)
    pltpu.sync_copy(buf_v, out_hbm.at[pl.ds(c * N_PAD + base, ROWS_SUB)])


def _t1_body(x_ref, w1_ref, b1_ref, cw_ref, degp_ref, p0_ref, p1_ref, p2_ref, dis_ref):
    degp = degp_ref[...]
    deg = jnp.reshape(degp[0] + degp[1], (N_PAD, 1))
    rid = lax.broadcasted_iota(jnp.int32, (N_PAD, 1), 0)
    dis = jnp.where((rid < N) & (deg > 0.0),
                    lax.rsqrt(jnp.maximum(deg, 1e-12)), 0.0)
    dis_ref[...] = dis
    h = jnp.maximum(
        jnp.dot(x_ref[...], w1_ref[...], preferred_element_type=jnp.float32)
        + b1_ref[...], 0.0)
    cw = cw_ref[...]
    p0_ref[...] = jnp.dot(h, cw[0] - cw[2], preferred_element_type=jnp.float32)
    p1_ref[...] = jnp.dot(h, cw[1], preferred_element_type=jnp.float32)
    p2_ref[...] = (dis * jnp.dot(h, cw[2], preferred_element_type=jnp.float32)).astype(jnp.bfloat16)


_t1 = pl.pallas_call(
    _t1_body,
    out_shape=[
        jax.ShapeDtypeStruct((N_PAD, D), jnp.float32),
        jax.ShapeDtypeStruct((N_PAD, D), jnp.float32),
        jax.ShapeDtypeStruct((N_PAD, D), jnp.bfloat16),
        jax.ShapeDtypeStruct((N_PAD, 1), jnp.float32),
    ],
)


def _tmid_body(p1_ref, q_ref, dis_ref, z_ref):
    q = q_ref[...].astype(jnp.float32)
    dis = dis_ref[...]
    z_ref[...] = (dis * p1_ref[...]
                  - 2.0 * dis * dis * (q[0] + q[1])).astype(jnp.bfloat16)


_tmid = pl.pallas_call(
    _tmid_body,
    out_shape=jax.ShapeDtypeStruct((N_PAD, D), jnp.bfloat16),
)


def _t3_body(p0_ref, q_ref, dis_ref, bc_ref, cw_ref, r0_ref, r1_ref, r2_ref):
    q = q_ref[...].astype(jnp.float32)
    dis = dis_ref[...]
    g = jnp.maximum(p0_ref[...] - dis * (q[0] + q[1]) + bc_ref[...], 0.0)
    cw = cw_ref[...]
    r0_ref[...] = jnp.dot(g, cw[0] - cw[2], preferred_element_type=jnp.float32)
    r1_ref[...] = jnp.dot(g, cw[1], preferred_element_type=jnp.float32)
    r2_ref[...] = (dis * jnp.dot(g, cw[2], preferred_element_type=jnp.float32)).astype(jnp.bfloat16)


_t3 = pl.pallas_call(
    _t3_body,
    out_shape=[
        jax.ShapeDtypeStruct((N_PAD, D), jnp.float32),
        jax.ShapeDtypeStruct((N_PAD, D), jnp.float32),
        jax.ShapeDtypeStruct((N_PAD, D), jnp.bfloat16),
    ],
)


def _t5_body(r0_ref, q_ref, dis_ref, bc_ref, w2_ref, b2_ref, out_ref):
    q = q_ref[...].astype(jnp.float32)
    dis = dis_ref[...]
    f = jnp.maximum(r0_ref[...] - dis * (q[0] + q[1]) + bc_ref[...], 0.0)
    logits = jnp.dot(f, w2_ref[...], preferred_element_type=jnp.float32) + b2_ref[...]
    col = lax.broadcasted_iota(jnp.int32, (N_PAD, HIDDEN), 1)
    mask = col < 2
    ml = jnp.where(mask, logits, -jnp.inf)
    m = jnp.max(ml, axis=1, keepdims=True)
    e = jnp.where(mask, jnp.exp(logits - m), 0.0)
    out_ref[...] = e / jnp.sum(e, axis=1, keepdims=True)


_t5 = pl.pallas_call(
    _t5_body,
    out_shape=jax.ShapeDtypeStruct((N_PAD, HIDDEN), jnp.float32),
)


@jax.jit
def kernel(x, edge_index, lin1_W, lin1_b, conv1_W, conv1_b, conv2_W, conv2_b,
           lin2_W, lin2_b):
    x_pad = jnp.zeros((N_PAD, D_IN), jnp.float32).at[:N].set(x)
    src = jnp.full((E_PAD,), N, jnp.int32).at[:E].set(
        edge_index[0].astype(jnp.int32)).reshape(R, CHUNK)
    dst = jnp.full((E_PAD,), N, jnp.int32).at[:E].set(
        edge_index[1].astype(jnp.int32)).reshape(R, CHUNK)
    w2_pad = jnp.zeros((D, HIDDEN), jnp.float32).at[:, :2].set(lin2_W)
    b2_pad = jnp.zeros((1, HIDDEN), jnp.float32).at[:, :2].set(lin2_b)

    degp = _sc_degree(src).reshape(NC, N_PAD)
    p0, p1, p2, dis = _t1(x_pad, lin1_W, lin1_b.reshape(1, HIDDEN), conv1_W, degp)

    q1 = _sc_spmm(p2, src, dst).reshape(NC, N_PAD, D)
    z1 = _tmid(p1, q1, dis)
    q2 = _sc_spmm(z1, src, dst).reshape(NC, N_PAD, D)
    r0, r1, r2 = _t3(p0, q2, dis, conv1_b.reshape(1, D), conv2_W)

    q3 = _sc_spmm(r2, src, dst).reshape(NC, N_PAD, D)
    z2 = _tmid(r1, q3, dis)
    q4 = _sc_spmm(z2, src, dst).reshape(NC, N_PAD, D)
    out = _t5(r0, q4, dis, conv2_b.reshape(1, D), w2_pad, b2_pad)

    return out[:N, :2]
